# baseline (speedup 1.0000x reference)
import jax
import jax.numpy as jnp
from jax import lax
from jax.experimental import pallas as pl
from jax.experimental.pallas import tpu as pltpu
from jax.experimental.pallas import tpu_sc as plsc

N_GRAPHS = 512
N_NODES = 100000
N_EDGES = 1600000
NODE_DIM = 128
EDGE_DIM = 16
HIDDEN = 64

NW = 32
L = 128

N_FULL = N_NODES // L
N_TAIL = N_NODES - N_FULL * L
N_CNT = N_FULL // NW
N_EXTRA = N_FULL - N_CNT * NW
N_MAX = N_CNT + 1

E_CHUNKS = N_EDGES // L
E_SLAB = 8
E_FULL = E_CHUNKS // E_SLAB
E_TCH = E_CHUNKS - E_FULL * E_SLAB
E_CNT = E_FULL // NW
E_EXTRA = E_FULL - E_CNT * NW
E_MAX = E_CNT + 1


def _node_phase(src_hbm, idxb, acc, buf0, buf1, sem0, sem1, start, cnt):
    @pl.when(cnt > 0)
    def _():
        pltpu.async_copy(src_hbm.at[pl.ds(start * L, L), :], buf0, sem0)

    @pl.when(cnt > 1)
    def _():
        pltpu.async_copy(src_hbm.at[pl.ds((start + 1) * L, L), :], buf1, sem1)

    def _pair(tp, _):
        for half, (buf, sem) in enumerate(((buf0, sem0), (buf1, sem1))):
            t = 2 * tp + half

            @pl.when(t < cnt)
            def _():
                pltpu.make_async_copy(
                    src_hbm.at[pl.ds((start + t) * L, L), :], buf, sem).wait()
                pltpu.sync_copy(buf, acc.at[idxb.at[pl.ds(t * L, L)]],
                                add=True)

                @pl.when(t + 2 < cnt)
                def _():
                    pltpu.async_copy(
                        src_hbm.at[pl.ds((start + t + 2) * L, L), :], buf, sem)
        return _

    lax.fori_loop(0, (N_MAX + 1) // 2, _pair, None)


def _edge_phase(src_hbm, idxb, acc, buf0, buf1, sem0, sem1, start, cnt):
    @pl.when(cnt > 0)
    def _():
        pltpu.async_copy(
            src_hbm.at[pl.ds(start * E_SLAB * L, E_SLAB * L), :], buf0, sem0)

    @pl.when(cnt > 1)
    def _():
        pltpu.async_copy(
            src_hbm.at[pl.ds((start + 1) * E_SLAB * L, E_SLAB * L), :],
            buf1, sem1)

    def _pair(tp, _):
        for half, (buf, sem) in enumerate(((buf0, sem0), (buf1, sem1))):
            t = 2 * tp + half

            @pl.when(t < cnt)
            def _():
                pltpu.make_async_copy(
                    src_hbm.at[pl.ds((start + t) * E_SLAB * L, E_SLAB * L), :],
                    buf, sem).wait()
                for j in range(E_SLAB):
                    pltpu.sync_copy(
                        buf.at[pl.ds(j * L, L), :],
                        acc.at[idxb.at[pl.ds((t * E_SLAB + j) * L, L)]],
                        add=True)

                @pl.when(t + 2 < cnt)
                def _():
                    pltpu.async_copy(
                        src_hbm.at[pl.ds((start + t + 2) * E_SLAB * L,
                                         E_SLAB * L), :], buf, sem)
        return _

    lax.fori_loop(0, (E_MAX + 1) // 2, _pair, None)


def _segsum_body(nodes_hbm, edges_hbm, nid_hbm, eid_hbm,
                 np_out, ep_out,
                 nbuf0, nbuf1, ebuf0, ebuf1, nidxb, eidxb, ntrows, ntidx,
                 zrow, sem0, sem1, sem2, sem3, nacc, eacc):
    c = lax.axis_index("c")
    s = lax.axis_index("s")
    wid = c * 16 + s

    n_start = wid * N_CNT + jnp.minimum(wid, N_EXTRA)
    n_cnt = N_CNT + jnp.where(wid < N_EXTRA, 1, 0)
    e_start = wid * E_CNT + jnp.minimum(wid, E_EXTRA)
    e_cnt = E_CNT + jnp.where(wid < E_EXTRA, 1, 0)

    pltpu.sync_copy(nid_hbm.at[pl.ds(n_start * L, N_MAX * L)], nidxb)
    pltpu.sync_copy(eid_hbm.at[pl.ds(e_start * E_SLAB * L, E_MAX * E_SLAB * L)],
                    eidxb)

    def _zero_row(i, _):
        zrow[pl.ds(i * 16, 16)] = jnp.zeros((16,), jnp.float32)
        return _
    lax.fori_loop(0, 8, _zero_row, None)
    base = s * (N_GRAPHS // 16)

    def _zero_nacc(i, _):
        pltpu.sync_copy(zrow, nacc.at[base + i])
        return _
    lax.fori_loop(0, N_GRAPHS // 16, _zero_nacc, None)

    def _zero_eacc(i, _):
        pltpu.sync_copy(zrow.at[pl.ds(0, EDGE_DIM)], eacc.at[base + i])
        return _
    lax.fori_loop(0, N_GRAPHS // 16, _zero_eacc, None)

    plsc.subcore_barrier()

    _node_phase(nodes_hbm, nidxb, nacc, nbuf0, nbuf1, sem0, sem1,
                n_start, n_cnt)

    @pl.when(wid == 30)
    def _():
        pltpu.sync_copy(nodes_hbm.at[pl.ds(N_FULL * L, N_TAIL), :], ntrows)
        pltpu.sync_copy(nid_hbm.at[pl.ds(N_FULL * L, N_TAIL)], ntidx)
        pltpu.sync_copy(ntrows, nacc.at[ntidx], add=True)

    _edge_phase(edges_hbm, eidxb, eacc, ebuf0, ebuf1, sem2, sem3,
                e_start, e_cnt)

    @pl.when(wid == NW - 1)
    def _():
        pltpu.sync_copy(edges_hbm.at[pl.ds(E_FULL * E_SLAB * L, E_TCH * L), :],
                        ebuf0.at[pl.ds(0, E_TCH * L), :])
        for j in range(E_TCH):
            pltpu.sync_copy(
                ebuf0.at[pl.ds(j * L, L), :],
                eacc.at[eidxb.at[pl.ds(((E_FULL - 1514) * E_SLAB + j) * L, L)]],
                add=True)

    plsc.subcore_barrier()

    rows = N_GRAPHS // 16
    pltpu.sync_copy(nacc.at[pl.ds(s * rows, rows), :],
                    np_out.at[c, pl.ds(s * rows, rows), :])
    pltpu.sync_copy(eacc.at[pl.ds(s * rows, rows), :],
                    ep_out.at[c, pl.ds(s * rows, rows), :])


def _mlp_body(np_ref, ep_ref, gg_ref, w1a_ref, w1b_ref, w1c_ref, b1_ref,
              w2_ref, b2_ref, out_ref):
    ns = np_ref[0] + np_ref[1]
    es = ep_ref[0] + ep_ref[1]
    x = (jnp.dot(gg_ref[...], w1a_ref[...], preferred_element_type=jnp.float32)
         + jnp.dot(ns, w1b_ref[...], preferred_element_type=jnp.float32)
         + jnp.dot(es, w1c_ref[...], preferred_element_type=jnp.float32)
         + b1_ref[...])
    h = jnp.maximum(x, 0.0)
    out_ref[...] = (jnp.dot(h, w2_ref[...], preferred_element_type=jnp.float32)
                    + b2_ref[...])


def kernel(nodes, batch, edges, batch_edges, graph_globals, W1, b1, W2, b2):
    bid = jnp.pad(batch.astype(jnp.int32), (0, N_MAX * L))
    eid = jnp.pad(batch_edges.astype(jnp.int32), (0, E_MAX * E_SLAB * L))

    mesh = plsc.VectorSubcoreMesh(core_axis_name="c", subcore_axis_name="s")
    segsum = pl.kernel(
        _segsum_body,
        out_type=[
            jax.ShapeDtypeStruct((2, N_GRAPHS, NODE_DIM), jnp.float32),
            jax.ShapeDtypeStruct((2, N_GRAPHS, EDGE_DIM), jnp.float32),
        ],
        mesh=mesh,
        compiler_params=pltpu.CompilerParams(use_tc_tiling_on_sc=False),
        scratch_types=[
            pltpu.VMEM((L, NODE_DIM), jnp.float32),
            pltpu.VMEM((L, NODE_DIM), jnp.float32),
            pltpu.VMEM((E_SLAB * L, EDGE_DIM), jnp.float32),
            pltpu.VMEM((E_SLAB * L, EDGE_DIM), jnp.float32),
            pltpu.VMEM((N_MAX * L,), jnp.int32),
            pltpu.VMEM((E_MAX * E_SLAB * L,), jnp.int32),
            pltpu.VMEM((N_TAIL, NODE_DIM), jnp.float32),
            pltpu.VMEM((N_TAIL,), jnp.int32),
            pltpu.VMEM((NODE_DIM,), jnp.float32),
            pltpu.SemaphoreType.DMA,
            pltpu.SemaphoreType.DMA,
            pltpu.SemaphoreType.DMA,
            pltpu.SemaphoreType.DMA,
            pltpu.VMEM_SHARED((N_GRAPHS, NODE_DIM), jnp.float32),
            pltpu.VMEM_SHARED((N_GRAPHS, EDGE_DIM), jnp.float32),
        ],
    )
    np_part, ep_part = segsum(nodes, edges, bid, eid)

    w1a = lax.slice(W1, (0, 0), (NODE_DIM, HIDDEN))
    w1b = lax.slice(W1, (NODE_DIM, 0), (2 * NODE_DIM, HIDDEN))
    w1c = lax.slice(W1, (2 * NODE_DIM, 0), (2 * NODE_DIM + EDGE_DIM, HIDDEN))

    out = pl.pallas_call(
        _mlp_body,
        out_shape=jax.ShapeDtypeStruct((N_GRAPHS, NODE_DIM), jnp.float32),
    )(np_part, ep_part, graph_globals, w1a, w1b, w1c,
      b1.reshape(1, HIDDEN), W2, b2.reshape(1, NODE_DIM))
    return out
